# baseline (speedup 1.0000x reference)
import functools

import jax
import jax.numpy as jnp
from jax import lax
from jax.experimental import pallas as pl
from jax.experimental.pallas import tpu as pltpu
from jax.experimental.pallas import tpu_sc as plsc

_BATCH = 4096
_HIST = 200
_D = 64
_NC = 2
_NS = 16
_NW = _NC * _NS
_BB = _BATCH // _NW
_L = 16

_mesh = plsc.VectorSubcoreMesh(core_axis_name="c", subcore_axis_name="s")


@functools.partial(
    pl.kernel,
    mesh=_mesh,
    out_type=jax.ShapeDtypeStruct((_HIST, 8, _NW, 8, _BB), jnp.float32),
    scratch_types=[
        pltpu.VMEM((_HIST, _BB), jnp.int32),
        pltpu.VMEM((_BB, _D), jnp.float32),
        pltpu.VMEM((_BB, _D), jnp.float32),
        pltpu.VMEM((8, 8, _BB), jnp.float32),
        pltpu.VMEM((8, 8, _BB), jnp.float32),
        pltpu.SemaphoreType.DMA,
        pltpu.SemaphoreType.DMA,
        pltpu.SemaphoreType.DMA,
        pltpu.SemaphoreType.DMA,
    ],
    compiler_params=pltpu.CompilerParams(
        use_tc_tiling_on_sc=False, needs_layout_passes=False
    ),
)
def _emb_kernel(table_hbm, idx_hbm, out_hbm, idx_v, rows0, rows1, t0, t1,
                sem_g0, sem_g1, sem_s0, sem_s1):
    wid = lax.axis_index("s") * _NC + lax.axis_index("c")
    rows = (rows0, rows1)
    tbuf = (t0, t1)
    sem_g = (sem_g0, sem_g1)
    sem_s = (sem_s0, sem_s1)

    pltpu.sync_copy(idx_hbm.at[wid], idx_v)

    def fire_gather(i, b):
        pltpu.async_copy(table_hbm.at[idx_v.at[i]], rows[b], sem_g[b])

    def drain_gather(b):
        pltpu.make_async_copy(
            table_hbm.at[pl.ds(0, _BB)], rows[b], sem_g[b]
        ).wait()

    def drain_stores(b):
        for dt in range(8):
            pltpu.make_async_copy(
                out_hbm.at[0, 0, 0], tbuf[b].at[dt], sem_s[b]
            ).wait()

    def transpose(b):
        src = rows[b]
        dst = tbuf[b]
        iota = lax.iota(jnp.int32, _L)
        for dt in range(8):
            for dr in range(8):
                col = jnp.full((_L,), 8 * dt + dr, jnp.int32)
                for c in range(_BB // _L):
                    v = plsc.load_gather(src, [iota + c * _L, col])
                    dst[dt, dr, pl.ds(c * _L, _L)] = v

    def fire_stores(i, b):
        for dt in range(8):
            pltpu.async_copy(tbuf[b].at[dt], out_hbm.at[i, dt, wid], sem_s[b])

    fire_gather(0, 0)

    def body(u, carry):
        for b in range(2):
            i = 2 * u + b
            drain_gather(b)

            @pl.when(i + 1 < _HIST)
            def _():
                fire_gather(i + 1, b ^ 1)

            @pl.when(i >= 2)
            def _():
                drain_stores(b)

            transpose(b)
            fire_stores(i, b)
        return carry

    lax.fori_loop(0, _HIST // 2, body, 0)
    drain_stores(0)
    drain_stores(1)


def kernel(item_seq, ID_embeddings):
    idx = (
        item_seq.astype(jnp.int32)
        .reshape(_NW, _BB, _HIST)
        .transpose(0, 2, 1)
    )
    five = _emb_kernel(ID_embeddings, idx)
    return five.transpose(2, 4, 0, 1, 3).reshape(_BATCH, _HIST, _D)


# speedup vs baseline: 1.0630x; 1.0630x over previous
import functools

import jax
import jax.numpy as jnp
from jax import lax
from jax.experimental import pallas as pl
from jax.experimental.pallas import tpu as pltpu
from jax.experimental.pallas import tpu_sc as plsc

_BATCH = 4096
_HIST = 200
_D = 64
_NC = 2
_NS = 16
_NW = _NC * _NS
_BB = _BATCH // _NW
_L = 16
_NSLOT = 4

_mesh = plsc.VectorSubcoreMesh(core_axis_name="c", subcore_axis_name="s")


@functools.partial(
    pl.kernel,
    mesh=_mesh,
    out_type=jax.ShapeDtypeStruct((_HIST, 8, _NW, 8, _BB), jnp.float32),
    scratch_types=[
        pltpu.VMEM((_HIST, _BB), jnp.int32),
        pltpu.VMEM((_BB, _D), jnp.float32),
        pltpu.VMEM((_BB, _D), jnp.float32),
        pltpu.VMEM((_BB, _D), jnp.float32),
        pltpu.VMEM((_BB, _D), jnp.float32),
        pltpu.VMEM((8, 8, _BB), jnp.float32),
        pltpu.VMEM((8, 8, _BB), jnp.float32),
        pltpu.SemaphoreType.DMA,
        pltpu.SemaphoreType.DMA,
        pltpu.SemaphoreType.DMA,
        pltpu.SemaphoreType.DMA,
        pltpu.SemaphoreType.DMA,
        pltpu.SemaphoreType.DMA,
    ],
    compiler_params=pltpu.CompilerParams(
        use_tc_tiling_on_sc=False, needs_layout_passes=False
    ),
)
def _emb_kernel(table_hbm, idx_hbm, out_hbm, idx_v, r0, r1, r2, r3, t0, t1,
                g0, g1, g2, g3, s0, s1):
    wid = lax.axis_index("s") * _NC + lax.axis_index("c")
    rows = (r0, r1, r2, r3)
    tbuf = (t0, t1)
    sem_g = (g0, g1, g2, g3)
    sem_s = (s0, s1)

    pltpu.sync_copy(idx_hbm.at[wid], idx_v)

    iota = lax.iota(jnp.int32, _L)
    rowids = [iota + c * _L for c in range(_BB // _L)]

    def fire_gather(i, slot):
        pltpu.async_copy(table_hbm.at[idx_v.at[i]], rows[slot], sem_g[slot])

    def drain_gather(slot):
        pltpu.make_async_copy(
            table_hbm.at[pl.ds(0, _BB)], rows[slot], sem_g[slot]
        ).wait()

    def drain_stores(b):
        for dt in range(8):
            pltpu.make_async_copy(
                out_hbm.at[0, 0, 0], tbuf[b].at[dt], sem_s[b]
            ).wait()

    def transpose(slot, b):
        src = rows[slot]
        dst = tbuf[b]

        def dt_body(dt, carry):
            for dr in range(8):
                col = jnp.full((_L,), 8 * dt + dr, jnp.int32)
                for c in range(_BB // _L):
                    v = plsc.load_gather(src, [rowids[c], col])
                    dst[dt, dr, pl.ds(c * _L, _L)] = v
            return carry

        lax.fori_loop(0, 8, dt_body, 0)

    def fire_stores(i, b):
        for dt in range(8):
            pltpu.async_copy(tbuf[b].at[dt], out_hbm.at[i, dt, wid], sem_s[b])

    for k in range(_NSLOT):
        fire_gather(k, k)

    def body(u, carry):
        for k in range(_NSLOT):
            i = _NSLOT * u + k
            b = k % 2
            drain_gather(k)

            @pl.when(i >= 2)
            def _():
                drain_stores(b)

            transpose(k, b)
            fire_stores(i, b)

            @pl.when(i + _NSLOT < _HIST)
            def _():
                fire_gather(i + _NSLOT, k)
        return carry

    lax.fori_loop(0, _HIST // _NSLOT, body, 0)
    drain_stores(0)
    drain_stores(1)


def kernel(item_seq, ID_embeddings):
    idx = (
        item_seq.astype(jnp.int32)
        .reshape(_NW, _BB, _HIST)
        .transpose(0, 2, 1)
    )
    five = _emb_kernel(ID_embeddings, idx)
    return five.transpose(2, 4, 0, 1, 3).reshape(_BATCH, _HIST, _D)


# speedup vs baseline: 2.0193x; 1.8996x over previous
import functools

import jax
import jax.numpy as jnp
from jax import lax
from jax.experimental import pallas as pl
from jax.experimental.pallas import tpu as pltpu
from jax.experimental.pallas import tpu_sc as plsc

_BATCH = 4096
_HIST = 200
_D = 64
_NC = 2
_NS = 16
_NW = _NC * _NS
_BB = _BATCH // _NW
_L = 16
_NSLOT = 4

_mesh = plsc.VectorSubcoreMesh(core_axis_name="c", subcore_axis_name="s")


@functools.partial(
    pl.kernel,
    mesh=_mesh,
    out_type=jax.ShapeDtypeStruct((_HIST, 8, _NW, 8, _BB), jnp.float32),
    scratch_types=[
        pltpu.VMEM((_HIST, _BB), jnp.int32),
        pltpu.VMEM((_BB, _D), jnp.float32),
        pltpu.VMEM((_BB, _D), jnp.float32),
        pltpu.VMEM((_BB, _D), jnp.float32),
        pltpu.VMEM((_BB, _D), jnp.float32),
        pltpu.VMEM((_D, _BB + 1), jnp.float32),
        pltpu.VMEM((_D, _BB + 1), jnp.float32),
        pltpu.SemaphoreType.DMA,
        pltpu.SemaphoreType.DMA,
        pltpu.SemaphoreType.DMA,
        pltpu.SemaphoreType.DMA,
        pltpu.SemaphoreType.DMA,
        pltpu.SemaphoreType.DMA,
    ],
    compiler_params=pltpu.CompilerParams(
        use_tc_tiling_on_sc=False, needs_layout_passes=False
    ),
)
def _emb_kernel(table_hbm, idx_hbm, out_hbm, idx_v, r0, r1, r2, r3, t0, t1,
                g0, g1, g2, g3, s0, s1):
    wid = lax.axis_index("s") * _NC + lax.axis_index("c")
    rows = (r0, r1, r2, r3)
    tbuf = (t0, t1)
    sem_g = (g0, g1, g2, g3)
    sem_s = (s0, s1)

    pltpu.sync_copy(idx_hbm.at[wid], idx_v)

    iota = lax.iota(jnp.int32, _L)
    dids = [iota + c * _L for c in range(_D // _L)]

    def fire_gather(i, slot):
        pltpu.async_copy(table_hbm.at[idx_v.at[i]], rows[slot], sem_g[slot])

    def drain_gather(slot):
        pltpu.make_async_copy(
            table_hbm.at[pl.ds(0, _BB)], rows[slot], sem_g[slot]
        ).wait()

    def drain_stores(b):
        for dt in range(8):
            pltpu.make_async_copy(
                out_hbm.at[0, 0, 0],
                tbuf[b].at[pl.ds(8 * dt, 8), pl.ds(0, _BB)],
                sem_s[b],
            ).wait()

    def transpose(slot, b):
        src = rows[slot]
        dst = tbuf[b]

        def bm_body(g, carry):
            for j in range(8):
                bm = 8 * g + j
                bm_vec = jnp.full((_L,), 0, jnp.int32) + bm
                for c in range(_D // _L):
                    v = src[bm, pl.ds(c * _L, _L)]
                    plsc.store_scatter(dst, [dids[c], bm_vec], v)
            return carry

        lax.fori_loop(0, _BB // 8, bm_body, 0)

    def fire_stores(i, b):
        for dt in range(8):
            pltpu.async_copy(
                tbuf[b].at[pl.ds(8 * dt, 8), pl.ds(0, _BB)],
                out_hbm.at[i, dt, wid],
                sem_s[b],
            )

    for k in range(_NSLOT):
        fire_gather(k, k)

    def body(u, carry):
        for k in range(_NSLOT):
            i = _NSLOT * u + k
            b = k % 2
            drain_gather(k)

            @pl.when(i >= 2)
            def _():
                drain_stores(b)

            transpose(k, b)
            fire_stores(i, b)

            @pl.when(i + _NSLOT < _HIST)
            def _():
                fire_gather(i + _NSLOT, k)
        return carry

    lax.fori_loop(0, _HIST // _NSLOT, body, 0)
    drain_stores(0)
    drain_stores(1)


def kernel(item_seq, ID_embeddings):
    idx = (
        item_seq.astype(jnp.int32)
        .reshape(_NW, _BB, _HIST)
        .transpose(0, 2, 1)
    )
    five = _emb_kernel(ID_embeddings, idx)
    return five.transpose(2, 4, 0, 1, 3).reshape(_BATCH, _HIST, _D)
